# Initial kernel scaffold; baseline (speedup 1.0000x reference)
import jax
import jax.numpy as jnp
from jax.experimental import pallas as pl


def kernel(per_atom_charge, atomic_subsystem_indices, electrostatic_pair_indices, electrostatic_d_ij):
    raise NotImplementedError("write your pallas kernel here")


# speedup vs baseline: 219.3029x; 219.3029x over previous
import functools
import jax
import jax.numpy as jnp
from jax import lax
from jax.experimental import pallas as pl
from jax.experimental.pallas import tpu as pltpu, tpu_sc as plsc

N_EDGES_K = 6400000
N_NODES_K = 100000
NUM_SYS_K = 1000

CHUNK = 2048
ROWS = 16
ROWW = 128
NCHUNK = N_EDGES_K // CHUNK
NW = 32
MAXK = (NCHUNK + NW - 1) // NW
EATOM_PAD = 100352
APT = EATOM_PAD // 16
ACC = 1024


def _rsqrt_f32(y):
    bits = lax.bitcast_convert_type(y, jnp.int32)
    bits = jnp.int32(0x5F3759DF) - lax.shift_right_arithmetic(bits, 1)
    r = lax.bitcast_convert_type(bits, jnp.float32)
    r = r * (jnp.float32(1.5) - jnp.float32(0.5) * y * r * r)
    r = r * (jnp.float32(1.5) - jnp.float32(0.5) * y * r * r)
    return r


def _coulomb_body(q_hbm, subsys_hbm, ii_hbm, jj_hbm, d_hbm, out_hbm,
                  q_v, ii2, jj_v, d_v, val_v, ebuf, sbuf, acc_v, eatom_sp):
    c = lax.axis_index("c")
    s = lax.axis_index("s")
    w = s * 2 + c

    zf = jnp.zeros((16,), jnp.float32)

    def zero_ebuf(i, carry):
        ebuf[pl.ds(i * 16, 16)] = zf
        return carry
    lax.fori_loop(0, APT // 16, zero_ebuf, 0)

    def zero_acc(i, carry):
        acc_v[pl.ds(i * 16, 16)] = zf
        return carry
    lax.fori_loop(0, ACC // 16, zero_acc, 0)

    pltpu.sync_copy(ebuf, eatom_sp.at[pl.ds(s * APT, APT)])
    pltpu.sync_copy(q_hbm, q_v)
    plsc.subcore_barrier()

    def chunk_step(k, carry):
        ch = w + k * NW

        @pl.when(ch < NCHUNK)
        def _():
            pltpu.sync_copy(ii_hbm.at[ch], ii2)
            pltpu.sync_copy(jj_hbm.at[ch], jj_v)
            pltpu.sync_copy(d_hbm.at[ch], d_v)

            def row_step(r, rc):
                base = pl.multiple_of(r * ROWW, ROWW)
                for u in range(ROWW // 16):
                    off = base + u * 16
                    ii = ii2[r, pl.ds(u * 16, 16)]
                    jj = jj_v[pl.ds(off, 16)]
                    d = d_v[pl.ds(off, 16)]
                    qi = plsc.load_gather(q_v, [ii])
                    qj = plsc.load_gather(q_v, [jj])
                    x = jnp.float32(2.0) * d
                    p = jnp.float32(1.0) + x * x * x * (
                        jnp.float32(-10.0) + x * (jnp.float32(15.0) - jnp.float32(6.0) * x))
                    phi = jnp.where(x < jnp.float32(1.0), p, jnp.float32(0.0))
                    y = d * d + jnp.float32(1.0)
                    rinv = _rsqrt_f32(y)
                    invd = jnp.float32(1.0) / d
                    chi = phi * rinv + (jnp.float32(1.0) - phi) * invd
                    e = qi * qj * chi
                    e = jnp.where(ii < jj, e, jnp.float32(0.0))
                    val_v[pl.ds(off, 16)] = e
                pltpu.sync_copy(val_v.at[pl.ds(base, ROWW)],
                                eatom_sp.at[ii2.at[r]], add=True)
                return rc
            lax.fori_loop(0, ROWS, row_step, 0)
        return carry
    lax.fori_loop(0, MAXK, chunk_step, 0)

    plsc.subcore_barrier()

    start = s * APT

    @pl.when(s < 15)
    def _():
        pltpu.sync_copy(eatom_sp.at[pl.ds(start, APT)], ebuf)
        pltpu.sync_copy(subsys_hbm.at[pl.ds(start, APT)], sbuf)

    @pl.when(s == 15)
    def _():
        pltpu.sync_copy(eatom_sp.at[pl.ds(15 * APT, N_NODES_K - 15 * APT)],
                        ebuf.at[pl.ds(0, N_NODES_K - 15 * APT)])
        pltpu.sync_copy(subsys_hbm.at[pl.ds(15 * APT, N_NODES_K - 15 * APT)],
                        sbuf.at[pl.ds(0, N_NODES_K - 15 * APT)])

    cnt = jnp.where(s == 15, N_NODES_K - 15 * APT, APT)
    lane = lax.iota(jnp.int32, 16)

    def p2_step(i, carry):
        base = i * 16
        sysv = sbuf[pl.ds(base, 16)]
        ev = ebuf[pl.ds(base, 16)]
        mask = (base + lane) < cnt
        plsc.addupdate_scatter(acc_v, [sysv], ev, mask=mask)
        return carry
    lax.fori_loop(0, APT // 16, p2_step, 0)

    pltpu.sync_copy(acc_v, out_hbm.at[w])


def kernel(per_atom_charge, atomic_subsystem_indices, electrostatic_pair_indices, electrostatic_d_ij):
    q = per_atom_charge.reshape(-1).astype(jnp.float32)
    subsys = atomic_subsystem_indices.astype(jnp.int32)
    ii3 = electrostatic_pair_indices[0].reshape(NCHUNK, ROWS, ROWW)
    jj2 = electrostatic_pair_indices[1].reshape(NCHUNK, CHUNK)
    d2 = electrostatic_d_ij.reshape(NCHUNK, CHUNK).astype(jnp.float32)

    mesh = plsc.VectorSubcoreMesh(core_axis_name="c", subcore_axis_name="s")
    run = functools.partial(
        pl.kernel,
        mesh=mesh,
        out_type=jax.ShapeDtypeStruct((NW, ACC), jnp.float32),
        scratch_types=[
            pltpu.VMEM((N_NODES_K,), jnp.float32),
            pltpu.VMEM((ROWS, ROWW), jnp.int32),
            pltpu.VMEM((CHUNK,), jnp.int32),
            pltpu.VMEM((CHUNK,), jnp.float32),
            pltpu.VMEM((CHUNK,), jnp.float32),
            pltpu.VMEM((APT,), jnp.float32),
            pltpu.VMEM((APT,), jnp.int32),
            pltpu.VMEM((ACC,), jnp.float32),
            pltpu.VMEM_SHARED((EATOM_PAD,), jnp.float32),
        ],
        compiler_params=pltpu.CompilerParams(needs_layout_passes=False),
    )(_coulomb_body)
    partial = run(q, subsys, ii3, jj2, d2)
    per_system = partial.sum(axis=0)[:NUM_SYS_K] * jnp.float32(138.96)
    return per_system[:, None]


# speedup vs baseline: 499.7133x; 2.2786x over previous
import functools
import jax
import jax.numpy as jnp
from jax import lax
from jax.experimental import pallas as pl
from jax.experimental.pallas import tpu as pltpu, tpu_sc as plsc

N_EDGES_K = 6400000
N_NODES_K = 100000
NUM_SYS_K = 1000

CHUNK = 2048
ROWS = 16
ROWW = 128
NCHUNK = N_EDGES_K // CHUNK
NW = 32
MAXK = (NCHUNK + NW - 1) // NW
EATOM_PAD = 100352
APT = EATOM_PAD // 16
ACC = 1024


def _rsqrt_f32(y):
    bits = lax.bitcast_convert_type(y, jnp.int32)
    bits = jnp.int32(0x5F3759DF) - lax.shift_right_arithmetic(bits, 1)
    r = lax.bitcast_convert_type(bits, jnp.float32)
    r = r * (jnp.float32(1.5) - jnp.float32(0.5) * y * r * r)
    r = r * (jnp.float32(1.5) - jnp.float32(0.5) * y * r * r)
    return r


def _coulomb_body(q_hbm, subsys_hbm, edges_hbm, out_hbm,
                  q_v, ibuf, val_v, ebuf, sbuf, acc_v, eatom_sp, scat_sem):
    c = lax.axis_index("c")
    s = lax.axis_index("s")
    w = s * 2 + c

    zf = jnp.zeros((16,), jnp.float32)

    def zero_ebuf(i, carry):
        ebuf[pl.ds(i * 16, 16)] = zf
        return carry
    lax.fori_loop(0, APT // 16, zero_ebuf, 0)

    def zero_acc(i, carry):
        acc_v[pl.ds(i * 16, 16)] = zf
        return carry
    lax.fori_loop(0, ACC // 16, zero_acc, 0)

    pltpu.sync_copy(ebuf, eatom_sp.at[pl.ds(s * APT, APT)])
    pltpu.sync_copy(q_hbm, q_v)
    plsc.subcore_barrier()

    def chunk_step(k, carry):
        ch = w + k * NW

        @pl.when(ch < NCHUNK)
        def _():
            pltpu.sync_copy(edges_hbm.at[ch], ibuf)

            def row_step(r, rc):
                for u in range(ROWW // 16):
                    ii = ibuf[0, r, pl.ds(u * 16, 16)]
                    jj = ibuf[1, r, pl.ds(u * 16, 16)]
                    d = lax.bitcast_convert_type(
                        ibuf[2, r, pl.ds(u * 16, 16)], jnp.float32)
                    qi = plsc.load_gather(q_v, [ii])
                    qj = plsc.load_gather(q_v, [jj])
                    x = jnp.float32(2.0) * d
                    p = jnp.float32(1.0) + x * x * x * (
                        jnp.float32(-10.0) + x * (jnp.float32(15.0) - jnp.float32(6.0) * x))
                    phi = jnp.where(x < jnp.float32(1.0), p, jnp.float32(0.0))
                    y = d * d + jnp.float32(1.0)
                    rinv = _rsqrt_f32(y)
                    invd = jnp.float32(1.0) / d
                    chi = phi * rinv + (jnp.float32(1.0) - phi) * invd
                    e = qi * qj * chi
                    e = jnp.where(ii < jj, e, jnp.float32(0.0))
                    val_v[r, pl.ds(u * 16, 16)] = e
                pltpu.async_copy(val_v.at[r], eatom_sp.at[ibuf.at[0, r]],
                                 scat_sem, add=True)
                return rc
            lax.fori_loop(0, ROWS, row_step, 0)

            def drain_step(r, rc):
                pltpu.make_async_copy(val_v.at[r],
                                      eatom_sp.at[ibuf.at[0, r]],
                                      scat_sem).wait()
                return rc
            lax.fori_loop(0, ROWS, drain_step, 0)
        return carry
    lax.fori_loop(0, MAXK, chunk_step, 0)

    plsc.subcore_barrier()

    start = s * APT

    @pl.when(s < 15)
    def _():
        pltpu.sync_copy(eatom_sp.at[pl.ds(start, APT)], ebuf)
        pltpu.sync_copy(subsys_hbm.at[pl.ds(start, APT)], sbuf)

    @pl.when(s == 15)
    def _():
        pltpu.sync_copy(eatom_sp.at[pl.ds(15 * APT, N_NODES_K - 15 * APT)],
                        ebuf.at[pl.ds(0, N_NODES_K - 15 * APT)])
        pltpu.sync_copy(subsys_hbm.at[pl.ds(15 * APT, N_NODES_K - 15 * APT)],
                        sbuf.at[pl.ds(0, N_NODES_K - 15 * APT)])

    cnt = jnp.where(s == 15, N_NODES_K - 15 * APT, APT)
    lane = lax.iota(jnp.int32, 16)

    def p2_step(i, carry):
        base = i * 16
        sysv = sbuf[pl.ds(base, 16)]
        ev = ebuf[pl.ds(base, 16)]
        mask = (base + lane) < cnt
        plsc.addupdate_scatter(acc_v, [sysv], ev, mask=mask)
        return carry
    lax.fori_loop(0, APT // 16, p2_step, 0)

    pltpu.sync_copy(acc_v, out_hbm.at[w])


def kernel(per_atom_charge, atomic_subsystem_indices, electrostatic_pair_indices, electrostatic_d_ij):
    q = per_atom_charge.reshape(-1).astype(jnp.float32)
    subsys = atomic_subsystem_indices.astype(jnp.int32)
    d_bits = lax.bitcast_convert_type(electrostatic_d_ij.astype(jnp.float32),
                                      jnp.int32)
    edges = jnp.stack(
        [electrostatic_pair_indices[0].reshape(NCHUNK, ROWS, ROWW),
         electrostatic_pair_indices[1].reshape(NCHUNK, ROWS, ROWW),
         d_bits.reshape(NCHUNK, ROWS, ROWW)], axis=1)

    mesh = plsc.VectorSubcoreMesh(core_axis_name="c", subcore_axis_name="s")
    run = functools.partial(
        pl.kernel,
        mesh=mesh,
        out_type=jax.ShapeDtypeStruct((NW, ACC), jnp.float32),
        scratch_types=[
            pltpu.VMEM((N_NODES_K,), jnp.float32),
            pltpu.VMEM((3, ROWS, ROWW), jnp.int32),
            pltpu.VMEM((ROWS, ROWW), jnp.float32),
            pltpu.VMEM((APT,), jnp.float32),
            pltpu.VMEM((APT,), jnp.int32),
            pltpu.VMEM((ACC,), jnp.float32),
            pltpu.VMEM_SHARED((EATOM_PAD,), jnp.float32),
            pltpu.SemaphoreType.DMA,
        ],
        compiler_params=pltpu.CompilerParams(needs_layout_passes=False),
    )(_coulomb_body)
    partial = run(q, subsys, edges)
    per_system = partial.sum(axis=0)[:NUM_SYS_K] * jnp.float32(138.96)
    return per_system[:, None]
